# Initial kernel scaffold; baseline (speedup 1.0000x reference)
import jax
import jax.numpy as jnp
from jax.experimental import pallas as pl


def kernel(uid, iid, u_feat, user_emb, user_bias, item_emb_mf, item_bias, feat_u, feat_i, mean, vae_mean, item_emb_lat):
    raise NotImplementedError("write your pallas kernel here")


# speedup vs baseline: 1.1954x; 1.1954x over previous
import functools

import jax
import jax.numpy as jnp
from jax import lax
from jax.experimental import pallas as pl
from jax.experimental.pallas import tpu as pltpu
from jax.experimental.pallas import tpu_sc as plsc

NUM_USERS = 100000
NUM_ITEMS = 100000
EMB = 64
N_FEAT = 4
FEAT_VOCAB = 1000
FEAT_DIM = 32
LATENT = 64
B = 16384

NC, NS, L = 2, 16, 16
NW = NC * NS
BPW = B // NW
CHUNK = 128
NCHUNK = BPW // CHUNK
GROUPS = CHUNK // L


def _sc_body(uid_h, iid_h, ufeat_h, ubias_h, ibias_h, uemb_h, iemb_h,
             featu_h, feati_h, mean_h, vmean_h, ilat_h, out_h,
             uid_v, iid_v, uf_v, fuidx_v, fiidx_v,
             U_v, I_v, Z_v, IL_v, FU_v, FI_v,
             bu_v, bi_v, mean_v, sbuf_v, out_v, sem):
    wid = lax.axis_index("s") * NC + lax.axis_index("c")
    base = wid * BPW
    lanes = lax.iota(jnp.int32, L)

    pltpu.sync_copy(mean_h, mean_v)

    def chunk_body(ci, _):
        cb = base + ci * CHUNK
        pltpu.sync_copy(uid_h.at[pl.ds(cb, CHUNK)], uid_v)
        pltpu.sync_copy(iid_h.at[pl.ds(cb, CHUNK)], iid_v)
        pltpu.sync_copy(ufeat_h.at[pl.ds(cb * N_FEAT, CHUNK * N_FEAT)], uf_v)

        def idx_body(j, _):
            iv = iid_v[pl.ds(j * L, L)]
            for f in range(N_FEAT):
                uv = plsc.load_gather(uf_v, [(j * L + lanes) * N_FEAT + f])
                fuidx_v[pl.ds(f * CHUNK + j * L, L)] = uv + f * FEAT_VOCAB
                fiidx_v[pl.ds(f * CHUNK + j * L, L)] = iv + f * NUM_ITEMS
            return 0

        lax.fori_loop(0, CHUNK // L, idx_body, 0)

        cps = [
            pltpu.async_copy(uemb_h.at[uid_v], U_v, sem),
            pltpu.async_copy(iemb_h.at[iid_v], I_v, sem),
            pltpu.async_copy(vmean_h.at[uid_v], Z_v, sem),
            pltpu.async_copy(ilat_h.at[iid_v], IL_v, sem),
            pltpu.async_copy(featu_h.at[fuidx_v], FU_v, sem),
            pltpu.async_copy(feati_h.at[fiidx_v], FI_v, sem),
            pltpu.async_copy(ubias_h.at[uid_v], bu_v, sem),
            pltpu.async_copy(ibias_h.at[iid_v], bi_v, sem),
        ]
        for cp in cps:
            cp.wait()

        def group_body(g, _):
            def row_body(j, _):
                r = g * L + j
                acc = U_v[r, pl.ds(0, L)] * I_v[r, pl.ds(0, L)]
                for k in range(1, EMB // L):
                    acc += U_v[r, pl.ds(k * L, L)] * I_v[r, pl.ds(k * L, L)]
                for k in range(LATENT // L):
                    acc += Z_v[r, pl.ds(k * L, L)] * IL_v[r, pl.ds(k * L, L)]
                for f in range(N_FEAT):
                    for k in range(FEAT_DIM // L):
                        acc += (FU_v[f * CHUNK + r, pl.ds(k * L, L)]
                                * FI_v[f * CHUNK + r, pl.ds(k * L, L)])
                sbuf_v[pl.ds(j * L, L)] = acc
                return 0

            lax.fori_loop(0, L, row_body, 0)

            t = bu_v[pl.ds(g * L, L)] + bi_v[pl.ds(g * L, L)] + mean_v[...]
            for d in range(L):
                t += plsc.load_gather(sbuf_v, [lanes * L + d])
            out_v[pl.ds(g * L, L)] = t
            return 0

        lax.fori_loop(0, GROUPS, group_body, 0)
        pltpu.sync_copy(out_v, out_h.at[pl.ds(cb, CHUNK)])
        return 0

    lax.fori_loop(0, NCHUNK, chunk_body, 0)


@jax.jit
def _sc_call(uid, iid, ufeat, ubias, ibias, uemb, iemb, featu, feati,
             mean16, vmean, ilat):
    mesh = plsc.VectorSubcoreMesh(core_axis_name="c", subcore_axis_name="s",
                                  num_cores=NC, num_subcores=NS)
    f = pl.kernel(
        _sc_body,
        out_type=jax.ShapeDtypeStruct((B,), jnp.float32),
        mesh=mesh,
        compiler_params=pltpu.CompilerParams(needs_layout_passes=False,
                                             use_tc_tiling_on_sc=False),
        scratch_types=[
            pltpu.VMEM((CHUNK,), jnp.int32),
            pltpu.VMEM((CHUNK,), jnp.int32),
            pltpu.VMEM((CHUNK * N_FEAT,), jnp.int32),
            pltpu.VMEM((CHUNK * N_FEAT,), jnp.int32),
            pltpu.VMEM((CHUNK * N_FEAT,), jnp.int32),
            pltpu.VMEM((CHUNK, EMB), jnp.float32),
            pltpu.VMEM((CHUNK, EMB), jnp.float32),
            pltpu.VMEM((CHUNK, LATENT), jnp.float32),
            pltpu.VMEM((CHUNK, LATENT), jnp.float32),
            pltpu.VMEM((CHUNK * N_FEAT, FEAT_DIM), jnp.float32),
            pltpu.VMEM((CHUNK * N_FEAT, FEAT_DIM), jnp.float32),
            pltpu.VMEM((CHUNK,), jnp.float32),
            pltpu.VMEM((CHUNK,), jnp.float32),
            pltpu.VMEM((L,), jnp.float32),
            pltpu.VMEM((L * L,), jnp.float32),
            pltpu.VMEM((CHUNK,), jnp.float32),
            pltpu.SemaphoreType.DMA,
        ],
    )
    return f(uid, iid, ufeat, ubias, ibias, uemb, iemb, featu, feati,
             mean16, vmean, ilat)


def kernel(uid, iid, u_feat, user_emb, user_bias, item_emb_mf, item_bias,
           feat_u, feat_i, mean, vae_mean, item_emb_lat):
    return _sc_call(
        uid, iid, u_feat.reshape(-1),
        user_bias.reshape(-1), item_bias.reshape(-1),
        user_emb, item_emb_mf,
        feat_u.reshape(-1, FEAT_DIM), feat_i.reshape(-1, FEAT_DIM),
        jnp.broadcast_to(mean, (L,)), vae_mean, item_emb_lat)
